# baseline (device time: 7397 ns/iter reference)
import jax
import jax.numpy as jnp
from jax import lax
from jax.experimental import pallas as pl
from jax.experimental.pallas import tpu as pltpu

N_DEV = 4
N_CHUNK = 2


def kernel(x):
    m_rows, n_cols = x.shape
    rows_c = m_rows // N_CHUNK

    def body(x_ref, out_ref, xv_ref, ov_ref, gs_ref,
             in_sems, out_sems, send_sems, recv_sems):
        my = lax.axis_index("i")

        barrier_sem = pltpu.get_barrier_semaphore()
        for d in range(1, N_DEV):
            peer = lax.rem(my + d, N_DEV)
            pl.semaphore_signal(
                barrier_sem, inc=1,
                device_id=(peer,), device_id_type=pl.DeviceIdType.MESH,
            )

        in_copies = []
        for c in range(N_CHUNK):
            rows = pl.ds(c * rows_c, rows_c)
            cp = pltpu.make_async_copy(x_ref.at[rows], xv_ref.at[rows],
                                       in_sems.at[c])
            cp.start()
            in_copies.append(cp)

        es = []
        sends = []
        for c in range(N_CHUNK):
            rows = pl.ds(c * rows_c, rows_c)
            in_copies[c].wait()
            xv = xv_ref[rows, :].astype(jnp.float32)
            e = jnp.exp(xv)
            es.append(e)
            s = jnp.sum(e, axis=1, keepdims=True)
            gs_ref[c, pl.ds(my, 1)] = s[:, 0][None, None, :]
            if c == 0:
                pl.semaphore_wait(barrier_sem, N_DEV - 1)
            for d in range(1, N_DEV):
                peer = lax.rem(my + d, N_DEV)
                rdma = pltpu.make_async_remote_copy(
                    src_ref=gs_ref.at[c, my],
                    dst_ref=gs_ref.at[c, my],
                    send_sem=send_sems.at[c, d - 1],
                    recv_sem=recv_sems.at[c, my],
                    device_id=(peer,),
                    device_id_type=pl.DeviceIdType.MESH,
                )
                rdma.start()
                sends.append(rdma)

        out_copies = []
        for c in range(N_CHUNK):
            rows = pl.ds(c * rows_c, rows_c)
            for d in range(1, N_DEV):
                peer = lax.rem(my + d, N_DEV)
                recv = pltpu.make_async_remote_copy(
                    src_ref=gs_ref.at[c, peer],
                    dst_ref=gs_ref.at[c, peer],
                    send_sem=send_sems.at[c, d - 1],
                    recv_sem=recv_sems.at[c, peer],
                    device_id=(peer,),
                    device_id_type=pl.DeviceIdType.MESH,
                )
                recv.wait_recv()
            gsum = jnp.sum(gs_ref[c, :, 0, :], axis=0)
            ov_ref[rows, :] = (es[c] * (1.0 / gsum)[:, None]).astype(
                jnp.bfloat16)
            cp = pltpu.make_async_copy(ov_ref.at[rows], out_ref.at[rows],
                                       out_sems.at[c])
            cp.start()
            out_copies.append(cp)

        for cp in out_copies:
            cp.wait()
        for rdma in sends:
            rdma.wait_send()

    return pl.pallas_call(
        body,
        out_shape=jax.ShapeDtypeStruct((m_rows, n_cols), jnp.bfloat16),
        in_specs=[pl.BlockSpec(memory_space=pl.ANY)],
        out_specs=pl.BlockSpec(memory_space=pl.ANY),
        scratch_shapes=[
            pltpu.VMEM((m_rows, n_cols), jnp.float32),
            pltpu.VMEM((m_rows, n_cols), jnp.bfloat16),
            pltpu.VMEM((N_CHUNK, N_DEV, 1, m_rows // N_CHUNK), jnp.float32),
            pltpu.SemaphoreType.DMA((N_CHUNK,)),
            pltpu.SemaphoreType.DMA((N_CHUNK,)),
            pltpu.SemaphoreType.DMA((N_CHUNK, N_DEV - 1)),
            pltpu.SemaphoreType.DMA((N_CHUNK, N_DEV)),
        ],
        compiler_params=pltpu.CompilerParams(collective_id=0),
    )(x)


# device time: 6896 ns/iter; 1.0727x vs baseline; 1.0727x over previous
import jax
import jax.numpy as jnp
from jax import lax
from jax.experimental import pallas as pl
from jax.experimental.pallas import tpu as pltpu

N_DEV = 4


def kernel(x):
    m_rows, n_cols = x.shape

    def body(x_ref, out_ref, gs_ref, send_sems, recv_sems):
        my = lax.axis_index("i")

        barrier_sem = pltpu.get_barrier_semaphore()
        for d in range(1, N_DEV):
            peer = lax.rem(my + d, N_DEV)
            pl.semaphore_signal(
                barrier_sem, inc=1,
                device_id=(peer,), device_id_type=pl.DeviceIdType.MESH,
            )

        e = jnp.exp(x_ref[:, :].astype(jnp.bfloat16))
        s = jnp.sum(e.astype(jnp.float32), axis=1, keepdims=True)
        s_row = s[:, 0][None, :]
        gs_ref[pl.ds(my, 1)] = s_row[None, :, :]

        pl.semaphore_wait(barrier_sem, N_DEV - 1)

        sends = []
        for d in range(1, N_DEV):
            peer = lax.rem(my + d, N_DEV)
            rdma = pltpu.make_async_remote_copy(
                src_ref=gs_ref.at[my],
                dst_ref=gs_ref.at[my],
                send_sem=send_sems.at[d - 1],
                recv_sem=recv_sems.at[my],
                device_id=(peer,),
                device_id_type=pl.DeviceIdType.MESH,
            )
            rdma.start()
            sends.append(rdma)

        for d in range(1, N_DEV):
            peer = lax.rem(my + d, N_DEV)
            recv = pltpu.make_async_remote_copy(
                src_ref=gs_ref.at[peer],
                dst_ref=gs_ref.at[peer],
                send_sem=send_sems.at[d - 1],
                recv_sem=recv_sems.at[peer],
                device_id=(peer,),
                device_id_type=pl.DeviceIdType.MESH,
            )
            recv.wait_recv()

        gsum = jnp.sum(gs_ref[:, 0, :], axis=0)
        inv = (1.0 / gsum).astype(jnp.bfloat16)
        out_ref[:, :] = e * inv[:, None]

        for rdma in sends:
            rdma.wait_send()

    return pl.pallas_call(
        body,
        out_shape=jax.ShapeDtypeStruct((m_rows, n_cols), jnp.bfloat16),
        in_specs=[pl.BlockSpec(memory_space=pltpu.VMEM)],
        out_specs=pl.BlockSpec(memory_space=pltpu.VMEM),
        scratch_shapes=[
            pltpu.VMEM((N_DEV, 1, m_rows), jnp.float32),
            pltpu.SemaphoreType.DMA((N_DEV - 1,)),
            pltpu.SemaphoreType.DMA((N_DEV,)),
        ],
        compiler_params=pltpu.CompilerParams(collective_id=0),
    )(x)
